# Initial kernel scaffold; baseline (speedup 1.0000x reference)
import jax
import jax.numpy as jnp
from jax.experimental import pallas as pl


def kernel(q_mu, q_var):
    raise NotImplementedError("write your pallas kernel here")


# speedup vs baseline: 142.7142x; 142.7142x over previous
import functools

import jax
import jax.numpy as jnp
from jax import lax
from jax.experimental import pallas as pl
from jax.experimental.pallas import tpu as pltpu
from jax.experimental.pallas import tpu_sc as plsc

N_OBJ = 100000
N_DIM = 128
ALPHA = 0.05
SENT = 1 << 20
BN = 2000
GRID = N_OBJ // BN
CH = BN
NCH = N_OBJ // CH
HIST_WORDS = N_OBJ + 32


def _tc_body(mu_ref, var_ref, eye_ref, bins_ref, c0_ref, k_ref):
    i = pl.program_id(0)
    mu = mu_ref[...]
    var = var_ref[...]
    z = (0.0 - mu) / (var * jnp.sqrt(jnp.float32(2.0)))
    p = 0.5 * (1.0 + lax.erf(z))
    x = p * jnp.float32(N_OBJ / ALPHA)
    be = jnp.clip(x.astype(jnp.int32), 0, N_OBJ)
    lo = jnp.clip(be - 2, 0, N_OBJ)
    b = lo
    for d in range(5):
        idx = lo + d
        valid = idx <= N_OBJ - 1
        tv = ((idx.astype(jnp.float32) + 1.0) / jnp.float32(N_OBJ)) * jnp.float32(ALPHA)
        b = b + jnp.where(valid & (tv < p), 1, 0)
    c0_part = jnp.sum(jnp.where(b == 0, 1, 0), axis=0, keepdims=True)
    k_part = jnp.sum(jnp.where(b < N_OBJ, 1, 0), axis=0, keepdims=True)
    bf = jnp.where(b == 0, SENT, b).astype(jnp.float32)
    bT = lax.dot_general(eye_ref[...], bf, (((1,), (1,)), ((), ())),
                         preferred_element_type=jnp.float32)
    bins_ref[...] = bT.astype(jnp.int32).reshape(1, N_DIM, BN)

    @pl.when(i == 0)
    def _():
        c0_ref[...] = c0_part
        k_ref[...] = k_part

    @pl.when(i != 0)
    def _():
        c0_ref[...] += c0_part
        k_ref[...] += k_part


def _tc_stage(q_mu, q_var):
    eye = jnp.eye(N_DIM, dtype=jnp.float32)
    return pl.pallas_call(
        _tc_body,
        grid=(GRID,),
        in_specs=[
            pl.BlockSpec((BN, N_DIM), lambda i: (i, 0)),
            pl.BlockSpec((BN, N_DIM), lambda i: (i, 0)),
            pl.BlockSpec((N_DIM, N_DIM), lambda i: (0, 0)),
        ],
        out_specs=[
            pl.BlockSpec((1, N_DIM, BN), lambda i: (i, 0, 0)),
            pl.BlockSpec((1, N_DIM), lambda i: (0, 0)),
            pl.BlockSpec((1, N_DIM), lambda i: (0, 0)),
        ],
        out_shape=[
            jax.ShapeDtypeStruct((GRID, N_DIM, BN), jnp.int32),
            jax.ShapeDtypeStruct((1, N_DIM), jnp.int32),
            jax.ShapeDtypeStruct((1, N_DIM), jnp.int32),
        ],
    )(q_mu, q_var, eye)


def _sc_body(bins_hbm, c0_hbm, k_hbm, out_hbm,
             kq, c0q, chunk, hist, res_v, sem):
    c = lax.axis_index("c")
    s = lax.axis_index("s")
    pltpu.sync_copy(k_hbm, kq)
    pltpu.sync_copy(c0_hbm, c0q)
    iota = lax.iota(jnp.int32, 16)
    zeros16 = jnp.zeros((16,), jnp.int32)
    ones16 = jnp.ones((16,), jnp.int32)
    res = jnp.zeros((16,), jnp.float32)
    big = jnp.int32(1 << 30)

    for j in range(4):
        r = c * 64 + s * 4 + j
        rvec = jnp.full((16,), r, jnp.int32)
        kvec = plsc.load_gather(kq, [rvec])
        c0vec = plsc.load_gather(c0q, [rvec])
        k_sc = jnp.max(kvec)

        nfull = k_sc // 16

        def zbody(i, carry):
            hist[pl.ds(i * 16, 16)] = zeros16
            return carry
        lax.fori_loop(0, nfull + 2, zbody, 0)

        for ci in range(NCH):
            pltpu.sync_copy(bins_hbm.at[ci, r], chunk)

            def sbody(i, carry):
                v = chunk[pl.ds(i * 16, 16)]
                m = v < kvec
                vsafe = jnp.where(m, v, 0)
                plsc.addupdate_scatter(hist, [vsafe], ones16, mask=m)
                return carry
            lax.fori_loop(0, CH // 16, sbody, 0)

        def scan_body(i, carry):
            cnt, first, run = carry
            h = hist[pl.ds(i * 16, 16)]
            cs = plsc.cumsum(h)
            pos = i * 16 + iota
            pred = (cs + run + c0vec) >= pos + 1
            cnt = cnt + jnp.sum(jnp.where(pred, 1, 0))
            first = jnp.minimum(first, jnp.min(jnp.where(pred, pos, big)))
            run = run + jnp.sum(h)
            return (cnt, first, run)
        cnt, first, run = lax.fori_loop(0, nfull, scan_body, (jnp.int32(0), big, jnp.int32(0)))
        h = hist[pl.ds(nfull * 16, 16)]
        cs = plsc.cumsum(h)
        pos = nfull * 16 + iota
        pred = ((cs + run + c0vec) >= pos + 1) & (pos < kvec)
        cnt = cnt + jnp.sum(jnp.where(pred, 1, 0))
        first = jnp.minimum(first, jnp.min(jnp.where(pred, pos, big)))

        imp = jnp.where(cnt > 0, (cnt + first).astype(jnp.float32), jnp.float32(0.0))
        res = jnp.where(iota == j, imp, res)

    res_v[...] = res
    pltpu.sync_copy(res_v, out_hbm.at[c * 16 + s])


@functools.partial(jax.jit, static_argnums=())
def _sc_stage(bins, c0, k):
    mesh = plsc.VectorSubcoreMesh(core_axis_name="c", subcore_axis_name="s")
    f = functools.partial(
        pl.kernel,
        out_type=jax.ShapeDtypeStruct((32, 16), jnp.float32),
        mesh=mesh,
        compiler_params=pltpu.CompilerParams(needs_layout_passes=False),
        scratch_types=[
            pltpu.VMEM((N_DIM,), jnp.int32),
            pltpu.VMEM((N_DIM,), jnp.int32),
            pltpu.VMEM((CH,), jnp.int32),
            pltpu.VMEM((HIST_WORDS,), jnp.int32),
            pltpu.VMEM((16,), jnp.float32),
            pltpu.SemaphoreType.DMA,
        ],
    )(_sc_body)
    out = f(bins, c0, k)
    return out[:, :4].reshape(N_DIM)


def kernel(q_mu, q_var):
    bins, c0, k = _tc_stage(q_mu, q_var)
    return _sc_stage(bins, c0.reshape(N_DIM), k.reshape(N_DIM))


# speedup vs baseline: 215.0501x; 1.5069x over previous
import functools

import jax
import jax.numpy as jnp
from jax import lax
from jax.experimental import pallas as pl
from jax.experimental.pallas import tpu as pltpu
from jax.experimental.pallas import tpu_sc as plsc

N_OBJ = 100000
N_DIM = 128
ALPHA = 0.05
SENT = 1 << 20
BN = 2000
GRID = N_OBJ // BN
CH = BN
NCH = N_OBJ // CH
SCAN_BLK = 128
HIST_WORDS = (N_OBJ // SCAN_BLK + 1) * SCAN_BLK


def _tc_body(mu_ref, var_ref, eye_ref, bins_ref, c0_ref, k_ref):
    i = pl.program_id(0)
    mu = mu_ref[...]
    var = var_ref[...]
    z = (0.0 - mu) / (var * jnp.sqrt(jnp.float32(2.0)))
    p = 0.5 * (1.0 + lax.erf(z))
    x = p * jnp.float32(N_OBJ / ALPHA)
    be = jnp.clip(x.astype(jnp.int32), 0, N_OBJ)
    lo = jnp.clip(be - 2, 0, N_OBJ)
    b = lo
    for d in range(5):
        idx = lo + d
        valid = idx <= N_OBJ - 1
        tv = ((idx.astype(jnp.float32) + 1.0) / jnp.float32(N_OBJ)) * jnp.float32(ALPHA)
        b = b + jnp.where(valid & (tv < p), 1, 0)
    c0_part = jnp.sum(jnp.where(b == 0, 1, 0), axis=0, keepdims=True)
    k_part = jnp.sum(jnp.where(b < N_OBJ, 1, 0), axis=0, keepdims=True)
    bf = jnp.where(b == 0, SENT, b).astype(jnp.float32)
    bT = lax.dot_general(eye_ref[...], bf, (((1,), (1,)), ((), ())),
                         preferred_element_type=jnp.float32)
    bins_ref[...] = bT.astype(jnp.int32).reshape(1, N_DIM, BN)

    @pl.when(i == 0)
    def _():
        c0_ref[...] = c0_part
        k_ref[...] = k_part

    @pl.when(i != 0)
    def _():
        c0_ref[...] += c0_part
        k_ref[...] += k_part


def _tc_stage(q_mu, q_var):
    eye = jnp.eye(N_DIM, dtype=jnp.float32)
    return pl.pallas_call(
        _tc_body,
        grid=(GRID,),
        in_specs=[
            pl.BlockSpec((BN, N_DIM), lambda i: (i, 0)),
            pl.BlockSpec((BN, N_DIM), lambda i: (i, 0)),
            pl.BlockSpec((N_DIM, N_DIM), lambda i: (0, 0)),
        ],
        out_specs=[
            pl.BlockSpec((1, N_DIM, BN), lambda i: (i, 0, 0)),
            pl.BlockSpec((1, N_DIM), lambda i: (0, 0)),
            pl.BlockSpec((1, N_DIM), lambda i: (0, 0)),
        ],
        out_shape=[
            jax.ShapeDtypeStruct((GRID, N_DIM, BN), jnp.int32),
            jax.ShapeDtypeStruct((1, N_DIM), jnp.int32),
            jax.ShapeDtypeStruct((1, N_DIM), jnp.int32),
        ],
    )(q_mu, q_var, eye)


def _sc_body(bins_hbm, c0_hbm, k_hbm, out_hbm,
             kq, c0q, ch0, ch1, hist, res_v, sem0, sem1):
    c = lax.axis_index("c")
    s = lax.axis_index("s")
    pltpu.sync_copy(k_hbm, kq)
    pltpu.sync_copy(c0_hbm, c0q)
    iota = lax.iota(jnp.int32, 16)
    zeros16 = jnp.zeros((16,), jnp.int32)
    ones16 = jnp.ones((16,), jnp.int32)
    res = jnp.zeros((16,), jnp.float32)
    big = jnp.int32(1 << 30)
    bigv = jnp.full((16,), big, jnp.int32)
    VPB = SCAN_BLK // 16

    def scatter_chunk(buf):
        def sbody(i, carry):
            v = buf[pl.ds(i * 16, 16)]
            m = v < carry
            vsafe = jnp.where(m, v, 0)
            plsc.addupdate_scatter(hist, [vsafe], ones16, mask=m)
            return carry
        return sbody

    for j in range(4):
        r = c * 64 + s * 4 + j
        rvec = jnp.full((16,), r, jnp.int32)
        kvec = plsc.load_gather(kq, [rvec])
        c0vec = plsc.load_gather(c0q, [rvec])
        k_sc = jnp.max(kvec)
        nblk = k_sc // SCAN_BLK + 1

        def zbody(i, carry):
            for d in range(VPB):
                hist[pl.ds(i * SCAN_BLK + d * 16, 16)] = zeros16
            return carry
        lax.fori_loop(0, nblk, zbody, 0)

        pltpu.async_copy(bins_hbm.at[0, r], ch0, sem0)
        pltpu.async_copy(bins_hbm.at[1, r], ch1, sem1)

        def cbody(i, carry):
            pltpu.make_async_copy(bins_hbm.at[2 * i, r], ch0, sem0).wait()
            lax.fori_loop(0, CH // 16, scatter_chunk(ch0), carry, unroll=5)
            pltpu.async_copy(bins_hbm.at[2 * i + 2, r], ch0, sem0)
            pltpu.make_async_copy(bins_hbm.at[2 * i + 1, r], ch1, sem1).wait()
            lax.fori_loop(0, CH // 16, scatter_chunk(ch1), carry, unroll=5)
            pltpu.async_copy(bins_hbm.at[2 * i + 3, r], ch1, sem1)
            return carry
        lax.fori_loop(0, NCH // 2 - 1, cbody, kvec)
        pltpu.make_async_copy(bins_hbm.at[NCH - 2, r], ch0, sem0).wait()
        lax.fori_loop(0, CH // 16, scatter_chunk(ch0), kvec, unroll=5)
        pltpu.make_async_copy(bins_hbm.at[NCH - 1, r], ch1, sem1).wait()
        lax.fori_loop(0, CH // 16, scatter_chunk(ch1), kvec, unroll=5)

        def scan_body(i, carry):
            cntv, firstv, runv = carry
            pos0 = i * SCAN_BLK + iota
            for d in range(VPB):
                h = hist[pl.ds(i * SCAN_BLK + d * 16, 16)]
                cs = plsc.cumsum(h)
                posv = pos0 + d * 16
                pred = (cs + runv + c0vec) >= posv + 1
                cntv = cntv + jnp.where(pred, 1, 0)
                firstv = jnp.minimum(firstv, jnp.where(pred, posv, big))
                runv = runv + jnp.sum(h)
            return (cntv, firstv, runv)
        cntv, firstv, _ = lax.fori_loop(
            0, nblk, scan_body, (zeros16, bigv, zeros16))
        cnt = jnp.sum(cntv)
        first = jnp.min(firstv)

        imp = jnp.where(cnt > 0, (cnt + first).astype(jnp.float32), jnp.float32(0.0))
        res = jnp.where(iota == j, imp, res)

    res_v[...] = res
    pltpu.sync_copy(res_v, out_hbm.at[c * 16 + s])


@functools.partial(jax.jit, static_argnums=())
def _sc_stage(bins, c0, k):
    mesh = plsc.VectorSubcoreMesh(core_axis_name="c", subcore_axis_name="s")
    f = functools.partial(
        pl.kernel,
        out_type=jax.ShapeDtypeStruct((32, 16), jnp.float32),
        mesh=mesh,
        compiler_params=pltpu.CompilerParams(needs_layout_passes=False),
        scratch_types=[
            pltpu.VMEM((N_DIM,), jnp.int32),
            pltpu.VMEM((N_DIM,), jnp.int32),
            pltpu.VMEM((CH,), jnp.int32),
            pltpu.VMEM((CH,), jnp.int32),
            pltpu.VMEM((HIST_WORDS,), jnp.int32),
            pltpu.VMEM((16,), jnp.float32),
            pltpu.SemaphoreType.DMA,
            pltpu.SemaphoreType.DMA,
        ],
    )(_sc_body)
    out = f(bins, c0, k)
    return out[:, :4].reshape(N_DIM)


def kernel(q_mu, q_var):
    bins, c0, k = _tc_stage(q_mu, q_var)
    return _sc_stage(bins, c0.reshape(N_DIM), k.reshape(N_DIM))


# speedup vs baseline: 246.4637x; 1.1461x over previous
import functools

import jax
import jax.numpy as jnp
from jax import lax
from jax.experimental import pallas as pl
from jax.experimental.pallas import tpu as pltpu
from jax.experimental.pallas import tpu_sc as plsc

N_OBJ = 100000
N_DIM = 128
ALPHA = 0.05
SENT = 1 << 20
BN = 2000
GRID = N_OBJ // BN
CH = BN
NCH = N_OBJ // CH
SCAN_BLK = 128
HIST_WORDS = (N_OBJ // SCAN_BLK + 1) * SCAN_BLK


def _tc_body(mu_ref, var_ref, eye_ref, bins_ref, c0_ref, k_ref):
    i = pl.program_id(0)
    mu = mu_ref[...]
    var = var_ref[...]
    z = (0.0 - mu) / (var * jnp.sqrt(jnp.float32(2.0)))
    p = 0.5 * (1.0 + lax.erf(z))
    x = p * jnp.float32(N_OBJ / ALPHA)
    be = x.astype(jnp.int32)
    bef = be.astype(jnp.float32)
    b = be - 1
    for d in range(3):
        tv = ((bef + jnp.float32(d)) / jnp.float32(N_OBJ)) * jnp.float32(ALPHA)
        b = b + jnp.where(tv < p, 1, 0)
    c0_part = jnp.sum(jnp.where(b <= 0, 1, 0), axis=0, keepdims=True)
    k_part = jnp.sum(jnp.where(b < N_OBJ, 1, 0), axis=0, keepdims=True)
    bf = jnp.where(b <= 0, SENT, jnp.minimum(b, N_OBJ)).astype(jnp.float32)
    bT = lax.dot_general(eye_ref[...], bf, (((1,), (1,)), ((), ())),
                         preferred_element_type=jnp.float32)
    bins_ref[...] = bT.astype(jnp.int32).reshape(1, N_DIM, BN)

    @pl.when(i == 0)
    def _():
        c0_ref[...] = c0_part
        k_ref[...] = k_part

    @pl.when(i != 0)
    def _():
        c0_ref[...] += c0_part
        k_ref[...] += k_part


def _tc_stage(q_mu, q_var):
    eye = jnp.eye(N_DIM, dtype=jnp.float32)
    return pl.pallas_call(
        _tc_body,
        grid=(GRID,),
        in_specs=[
            pl.BlockSpec((BN, N_DIM), lambda i: (i, 0)),
            pl.BlockSpec((BN, N_DIM), lambda i: (i, 0)),
            pl.BlockSpec((N_DIM, N_DIM), lambda i: (0, 0)),
        ],
        out_specs=[
            pl.BlockSpec((1, N_DIM, BN), lambda i: (i, 0, 0)),
            pl.BlockSpec((1, N_DIM), lambda i: (0, 0)),
            pl.BlockSpec((1, N_DIM), lambda i: (0, 0)),
        ],
        out_shape=[
            jax.ShapeDtypeStruct((GRID, N_DIM, BN), jnp.int32),
            jax.ShapeDtypeStruct((1, N_DIM), jnp.int32),
            jax.ShapeDtypeStruct((1, N_DIM), jnp.int32),
        ],
    )(q_mu, q_var, eye)


def _sc_body(bins_hbm, c0_hbm, k_hbm, out_hbm,
             kq, c0q, ch0, ch1, hist, res_v, sem0, sem1):
    c = lax.axis_index("c")
    s = lax.axis_index("s")
    pltpu.sync_copy(k_hbm, kq)
    pltpu.sync_copy(c0_hbm, c0q)
    iota = lax.iota(jnp.int32, 16)
    zeros16 = jnp.zeros((16,), jnp.int32)
    ones16 = jnp.ones((16,), jnp.int32)
    res = jnp.zeros((16,), jnp.float32)
    big = jnp.int32(1 << 30)
    bigv = jnp.full((16,), big, jnp.int32)
    VPB = SCAN_BLK // 16

    def scatter_chunk(buf):
        def sbody(i, carry):
            v = buf[pl.ds(i * 16, 16)]
            m = v < carry
            plsc.addupdate_scatter(hist, [v], ones16, mask=m)
            return carry
        return sbody

    prev_zeroed = jnp.int32(0)
    for j in range(4):
        r = c * 64 + s * 4 + j
        rvec = jnp.full((16,), r, jnp.int32)
        kvec = plsc.load_gather(kq, [rvec])
        c0vec = plsc.load_gather(c0q, [rvec])
        k_sc = jnp.max(kvec)
        nblk = k_sc // SCAN_BLK + 1

        def zbody(i, carry):
            for d in range(VPB):
                hist[pl.ds(i * SCAN_BLK + d * 16, 16)] = zeros16
            return carry
        lax.fori_loop(prev_zeroed, nblk, zbody, 0)
        prev_zeroed = nblk

        pltpu.async_copy(bins_hbm.at[0, r], ch0, sem0)
        pltpu.async_copy(bins_hbm.at[1, r], ch1, sem1)

        def cbody(i, carry):
            pltpu.make_async_copy(bins_hbm.at[2 * i, r], ch0, sem0).wait()
            lax.fori_loop(0, CH // 16, scatter_chunk(ch0), carry, unroll=5)
            pltpu.async_copy(bins_hbm.at[2 * i + 2, r], ch0, sem0)
            pltpu.make_async_copy(bins_hbm.at[2 * i + 1, r], ch1, sem1).wait()
            lax.fori_loop(0, CH // 16, scatter_chunk(ch1), carry, unroll=5)
            pltpu.async_copy(bins_hbm.at[2 * i + 3, r], ch1, sem1)
            return carry
        lax.fori_loop(0, NCH // 2 - 1, cbody, kvec)
        pltpu.make_async_copy(bins_hbm.at[NCH - 2, r], ch0, sem0).wait()
        lax.fori_loop(0, CH // 16, scatter_chunk(ch0), kvec, unroll=5)
        pltpu.make_async_copy(bins_hbm.at[NCH - 1, r], ch1, sem1).wait()
        lax.fori_loop(0, CH // 16, scatter_chunk(ch1), kvec, unroll=5)

        def scan_body(i, carry):
            cntv, firstv, runv = carry
            pos0 = i * SCAN_BLK + iota
            hs = [hist[pl.ds(i * SCAN_BLK + d * 16, 16)] for d in range(VPB)]
            css = [plsc.cumsum(h) for h in hs]
            sums = [jnp.sum(h) for h in hs]
            for d in range(VPB):
                hist[pl.ds(i * SCAN_BLK + d * 16, 16)] = zeros16
            offs = [jnp.int32(0)]
            acc = jnp.int32(0)
            for d in range(VPB - 1):
                acc = acc + sums[d]
                offs.append(acc)
            total = acc + sums[VPB - 1]
            for d in range(VPB):
                posv = pos0 + d * 16
                pred = (css[d] + (runv + offs[d]) + c0vec) >= posv + 1
                cntv = cntv + jnp.where(pred, 1, 0)
                firstv = jnp.minimum(firstv, jnp.where(pred, posv, big))
            runv = runv + total
            return (cntv, firstv, runv)
        cntv, firstv, _ = lax.fori_loop(
            0, nblk, scan_body, (zeros16, bigv, zeros16))
        cnt = jnp.sum(cntv)
        first = jnp.min(firstv)

        imp = jnp.where(cnt > 0, (cnt + first).astype(jnp.float32), jnp.float32(0.0))
        res = jnp.where(iota == j, imp, res)

    res_v[...] = res
    pltpu.sync_copy(res_v, out_hbm.at[c * 16 + s])


@functools.partial(jax.jit, static_argnums=())
def _sc_stage(bins, c0, k):
    mesh = plsc.VectorSubcoreMesh(core_axis_name="c", subcore_axis_name="s")
    f = functools.partial(
        pl.kernel,
        out_type=jax.ShapeDtypeStruct((32, 16), jnp.float32),
        mesh=mesh,
        compiler_params=pltpu.CompilerParams(needs_layout_passes=False),
        scratch_types=[
            pltpu.VMEM((N_DIM,), jnp.int32),
            pltpu.VMEM((N_DIM,), jnp.int32),
            pltpu.VMEM((CH,), jnp.int32),
            pltpu.VMEM((CH,), jnp.int32),
            pltpu.VMEM((HIST_WORDS,), jnp.int32),
            pltpu.VMEM((16,), jnp.float32),
            pltpu.SemaphoreType.DMA,
            pltpu.SemaphoreType.DMA,
        ],
    )(_sc_body)
    out = f(bins, c0, k)
    return out[:, :4].reshape(N_DIM)


def kernel(q_mu, q_var):
    bins, c0, k = _tc_stage(q_mu, q_var)
    return _sc_stage(bins, c0.reshape(N_DIM), k.reshape(N_DIM))


# speedup vs baseline: 312.2206x; 1.2668x over previous
import functools

import jax
import jax.numpy as jnp
from jax import lax
from jax.experimental import pallas as pl
from jax.experimental.pallas import tpu as pltpu
from jax.experimental.pallas import tpu_sc as plsc

N_OBJ = 100000
N_DIM = 128
ALPHA = 0.05
SENT = 1 << 20
BN = 2000
GRID = N_OBJ // BN
CH = BN
NCH = N_OBJ // CH
SCAN_BLK = 128
HIST_WORDS = (N_OBJ // SCAN_BLK + 1) * SCAN_BLK


def _tc_body(mu_ref, var_ref, eye_ref, bins_ref, c0_ref, k_ref):
    i = pl.program_id(0)
    mu = mu_ref[...]
    var = var_ref[...]
    z = (0.0 - mu) / (var * jnp.sqrt(jnp.float32(2.0)))
    p = 0.5 * (1.0 + lax.erf(z))
    x = p * jnp.float32(N_OBJ / ALPHA)
    be = x.astype(jnp.int32)
    bef = be.astype(jnp.float32)
    b = be - 1
    for d in range(3):
        tv = ((bef + jnp.float32(d)) / jnp.float32(N_OBJ)) * jnp.float32(ALPHA)
        b = b + jnp.where(tv < p, 1, 0)
    c0_part = jnp.sum(jnp.where(b <= 0, 1, 0), axis=0, keepdims=True)
    k_part = jnp.sum(jnp.where(b < N_OBJ, 1, 0), axis=0, keepdims=True)
    bf = jnp.where(b <= 0, SENT, jnp.minimum(b, N_OBJ)).astype(jnp.float32)
    bT = lax.dot_general(eye_ref[...], bf, (((1,), (1,)), ((), ())),
                         preferred_element_type=jnp.float32)
    bins_ref[...] = bT.astype(jnp.int32).reshape(1, N_DIM, BN)

    @pl.when(i == 0)
    def _():
        c0_ref[...] = c0_part
        k_ref[...] = k_part

    @pl.when(i != 0)
    def _():
        c0_ref[...] += c0_part
        k_ref[...] += k_part


def _tc_stage(q_mu, q_var):
    eye = jnp.eye(N_DIM, dtype=jnp.float32)
    return pl.pallas_call(
        _tc_body,
        grid=(GRID,),
        in_specs=[
            pl.BlockSpec((BN, N_DIM), lambda i: (i, 0)),
            pl.BlockSpec((BN, N_DIM), lambda i: (i, 0)),
            pl.BlockSpec((N_DIM, N_DIM), lambda i: (0, 0)),
        ],
        out_specs=[
            pl.BlockSpec((1, N_DIM, BN), lambda i: (i, 0, 0)),
            pl.BlockSpec((1, N_DIM), lambda i: (0, 0)),
            pl.BlockSpec((1, N_DIM), lambda i: (0, 0)),
        ],
        out_shape=[
            jax.ShapeDtypeStruct((GRID, N_DIM, BN), jnp.int32),
            jax.ShapeDtypeStruct((1, N_DIM), jnp.int32),
            jax.ShapeDtypeStruct((1, N_DIM), jnp.int32),
        ],
    )(q_mu, q_var, eye)


def _sc_body(bins_hbm, c0_hbm, k_hbm, out_hbm,
             kq, c0q, ch0, ch1, hist, res_v, sem0, sem1):
    c = lax.axis_index("c")
    s = lax.axis_index("s")
    pltpu.sync_copy(k_hbm, kq)
    pltpu.sync_copy(c0_hbm, c0q)
    iota = lax.iota(jnp.int32, 16)
    zeros16 = jnp.zeros((16,), jnp.int32)
    ones16 = jnp.ones((16,), jnp.int32)
    res = jnp.zeros((16,), jnp.float32)
    big = jnp.int32(1 << 30)
    bigv = jnp.full((16,), big, jnp.int32)
    VPB = SCAN_BLK // 16

    prev_zeroed = jnp.int32(0)
    for j in range(4):
        r = c * 64 + s * 4 + j
        rvec = jnp.full((16,), r, jnp.int32)
        kvec = plsc.load_gather(kq, [rvec])
        c0vec = plsc.load_gather(c0q, [rvec])
        k_sc = jnp.max(kvec)
        nblk = k_sc // SCAN_BLK + 1

        def zbody(i, carry):
            for d in range(VPB):
                hist[pl.ds(i * SCAN_BLK + d * 16, 16)] = zeros16
            return carry
        lax.fori_loop(prev_zeroed, nblk, zbody, 0)
        prev_zeroed = nblk

        def scatter_chunk(buf):
            v1 = buf[pl.ds(0, 16)]
            m1 = v1 < kvec
            v2 = buf[pl.ds(16, 16)]

            def sbody(i, carry):
                va, ma, vb = carry
                plsc.addupdate_scatter(hist, [va], ones16, mask=ma)
                mb = vb < kvec
                vc = buf[pl.ds(i * 16 + 32, 16)]
                return (vb, mb, vc)
            va, ma, vb = lax.fori_loop(
                0, CH // 16 - 2, sbody, (v1, m1, v2), unroll=5)
            plsc.addupdate_scatter(hist, [va], ones16, mask=ma)
            plsc.addupdate_scatter(hist, [vb], ones16, mask=vb < kvec)

        pltpu.async_copy(bins_hbm.at[0, r], ch0, sem0)
        pltpu.async_copy(bins_hbm.at[1, r], ch1, sem1)

        def cbody(i, carry):
            pltpu.make_async_copy(bins_hbm.at[2 * i, r], ch0, sem0).wait()
            scatter_chunk(ch0)
            pltpu.async_copy(bins_hbm.at[2 * i + 2, r], ch0, sem0)
            pltpu.make_async_copy(bins_hbm.at[2 * i + 1, r], ch1, sem1).wait()
            scatter_chunk(ch1)
            pltpu.async_copy(bins_hbm.at[2 * i + 3, r], ch1, sem1)
            return carry
        lax.fori_loop(0, NCH // 2 - 1, cbody, jnp.int32(0))
        pltpu.make_async_copy(bins_hbm.at[NCH - 2, r], ch0, sem0).wait()
        scatter_chunk(ch0)
        pltpu.make_async_copy(bins_hbm.at[NCH - 1, r], ch1, sem1).wait()
        scatter_chunk(ch1)

        def scan_body(i, carry):
            cntv, firstv, runv = carry
            pos0 = i * SCAN_BLK + iota
            hs = [hist[pl.ds(i * SCAN_BLK + d * 16, 16)] for d in range(VPB)]
            css = [plsc.cumsum(h) for h in hs]
            sums = [jnp.sum(h) for h in hs]
            for d in range(VPB):
                hist[pl.ds(i * SCAN_BLK + d * 16, 16)] = zeros16
            offs = [jnp.int32(0)]
            acc = jnp.int32(0)
            for d in range(VPB - 1):
                acc = acc + sums[d]
                offs.append(acc)
            total = acc + sums[VPB - 1]
            for d in range(VPB):
                posv = pos0 + d * 16
                pred = (css[d] + (runv + offs[d]) + c0vec) >= posv + 1
                cntv = cntv + jnp.where(pred, 1, 0)
                firstv = jnp.minimum(firstv, jnp.where(pred, posv, big))
            runv = runv + total
            return (cntv, firstv, runv)
        cntv, firstv, _ = lax.fori_loop(
            0, nblk, scan_body, (zeros16, bigv, zeros16))
        cnt = jnp.sum(cntv)
        first = jnp.min(firstv)

        imp = jnp.where(cnt > 0, (cnt + first).astype(jnp.float32), jnp.float32(0.0))
        res = jnp.where(iota == j, imp, res)

    res_v[...] = res
    pltpu.sync_copy(res_v, out_hbm.at[c * 16 + s])


@functools.partial(jax.jit, static_argnums=())
def _sc_stage(bins, c0, k):
    mesh = plsc.VectorSubcoreMesh(core_axis_name="c", subcore_axis_name="s")
    f = functools.partial(
        pl.kernel,
        out_type=jax.ShapeDtypeStruct((32, 16), jnp.float32),
        mesh=mesh,
        compiler_params=pltpu.CompilerParams(needs_layout_passes=False),
        scratch_types=[
            pltpu.VMEM((N_DIM,), jnp.int32),
            pltpu.VMEM((N_DIM,), jnp.int32),
            pltpu.VMEM((CH,), jnp.int32),
            pltpu.VMEM((CH,), jnp.int32),
            pltpu.VMEM((HIST_WORDS,), jnp.int32),
            pltpu.VMEM((16,), jnp.float32),
            pltpu.SemaphoreType.DMA,
            pltpu.SemaphoreType.DMA,
        ],
    )(_sc_body)
    out = f(bins, c0, k)
    return out[:, :4].reshape(N_DIM)


def kernel(q_mu, q_var):
    bins, c0, k = _tc_stage(q_mu, q_var)
    return _sc_stage(bins, c0.reshape(N_DIM), k.reshape(N_DIM))


# speedup vs baseline: 326.2331x; 1.0449x over previous
import functools

import jax
import jax.numpy as jnp
from jax import lax
from jax.experimental import pallas as pl
from jax.experimental.pallas import tpu as pltpu
from jax.experimental.pallas import tpu_sc as plsc

N_OBJ = 100000
N_DIM = 128
ALPHA = 0.05
SENT = 1 << 20
BN = 2000
GRID = N_OBJ // BN
CH = BN
NCH = N_OBJ // CH
SCAN_BLK = 128
HIST_WORDS = (N_OBJ // SCAN_BLK + 1) * SCAN_BLK


def _tc_body(mu_ref, var_ref, eye_ref, bins_ref, c0_ref, k_ref):
    i = pl.program_id(0)
    mu = mu_ref[...]
    var = var_ref[...]
    z = (0.0 - mu) / (var * jnp.sqrt(jnp.float32(2.0)))
    p = 0.5 * (1.0 + lax.erf(z))
    x = p * jnp.float32(N_OBJ / ALPHA)
    be = x.astype(jnp.int32)
    bef = be.astype(jnp.float32)
    b = be - 1
    for d in range(3):
        tv = ((bef + jnp.float32(d)) / jnp.float32(N_OBJ)) * jnp.float32(ALPHA)
        b = b + jnp.where(tv < p, 1, 0)
    c0_part = jnp.sum(jnp.where(b <= 0, 1, 0), axis=0, keepdims=True)
    k_part = jnp.sum(jnp.where(b < N_OBJ, 1, 0), axis=0, keepdims=True)
    bf = jnp.where(b <= 0, SENT, jnp.minimum(b, N_OBJ)).astype(jnp.float32)
    bT = lax.dot_general(eye_ref[...], bf, (((1,), (1,)), ((), ())),
                         preferred_element_type=jnp.float32)
    bins_ref[...] = bT.astype(jnp.int32).reshape(1, N_DIM, BN)

    @pl.when(i == 0)
    def _():
        c0_ref[...] = c0_part
        k_ref[...] = k_part

    @pl.when(i != 0)
    def _():
        c0_ref[...] += c0_part
        k_ref[...] += k_part


def _tc_stage(q_mu, q_var):
    eye = jnp.eye(N_DIM, dtype=jnp.float32)
    return pl.pallas_call(
        _tc_body,
        grid=(GRID,),
        in_specs=[
            pl.BlockSpec((BN, N_DIM), lambda i: (i, 0)),
            pl.BlockSpec((BN, N_DIM), lambda i: (i, 0)),
            pl.BlockSpec((N_DIM, N_DIM), lambda i: (0, 0)),
        ],
        out_specs=[
            pl.BlockSpec((1, N_DIM, BN), lambda i: (i, 0, 0)),
            pl.BlockSpec((1, N_DIM), lambda i: (0, 0)),
            pl.BlockSpec((1, N_DIM), lambda i: (0, 0)),
        ],
        out_shape=[
            jax.ShapeDtypeStruct((GRID, N_DIM, BN), jnp.int32),
            jax.ShapeDtypeStruct((1, N_DIM), jnp.int32),
            jax.ShapeDtypeStruct((1, N_DIM), jnp.int32),
        ],
    )(q_mu, q_var, eye)


def _sc_body(bins_hbm, c0_hbm, k_hbm, out_hbm,
             kq, c0q, ch0, ch1, hist, res_v, sem0, sem1):
    c = lax.axis_index("c")
    s = lax.axis_index("s")
    pltpu.sync_copy(k_hbm, kq)
    pltpu.sync_copy(c0_hbm, c0q)
    iota = lax.iota(jnp.int32, 16)
    zeros16 = jnp.zeros((16,), jnp.int32)
    ones16 = jnp.ones((16,), jnp.int32)
    res = jnp.zeros((16,), jnp.float32)
    big = jnp.int32(1 << 30)
    bigv = jnp.full((16,), big, jnp.int32)
    VPB = SCAN_BLK // 16

    prev_zeroed = jnp.int32(0)
    for j in range(4):
        r = c * 64 + s * 4 + j
        rvec = jnp.full((16,), r, jnp.int32)
        kvec = plsc.load_gather(kq, [rvec])
        c0vec = plsc.load_gather(c0q, [rvec])
        k_sc = jnp.max(kvec)
        nblk = k_sc // SCAN_BLK + 1

        def zbody(i, carry):
            for d in range(VPB):
                hist[pl.ds(i * SCAN_BLK + d * 16, 16)] = zeros16
            return carry
        lax.fori_loop(prev_zeroed, nblk, zbody, 0)
        prev_zeroed = nblk

        def scatter_chunk(buf):
            v1 = buf[pl.ds(0, 16)]
            m1 = v1 < kvec
            v2 = buf[pl.ds(16, 16)]
            m2 = v2 < kvec
            v3 = buf[pl.ds(32, 16)]

            def sbody(i, carry):
                va, ma, vb, mb, vc = carry
                plsc.addupdate_scatter(hist, [va], ones16, mask=ma)
                mc = vc < kvec
                vd = buf[pl.ds(i * 16 + 48, 16)]
                return (vb, mb, vc, mc, vd)
            va, ma, vb, mb, vc = lax.fori_loop(
                0, CH // 16 - 3, sbody, (v1, m1, v2, m2, v3), unroll=5)
            plsc.addupdate_scatter(hist, [va], ones16, mask=ma)
            plsc.addupdate_scatter(hist, [vb], ones16, mask=mb)
            plsc.addupdate_scatter(hist, [vc], ones16, mask=vc < kvec)

        pltpu.async_copy(bins_hbm.at[0, r], ch0, sem0)
        pltpu.async_copy(bins_hbm.at[1, r], ch1, sem1)

        def cbody(i, carry):
            pltpu.make_async_copy(bins_hbm.at[2 * i, r], ch0, sem0).wait()
            scatter_chunk(ch0)
            pltpu.async_copy(bins_hbm.at[2 * i + 2, r], ch0, sem0)
            pltpu.make_async_copy(bins_hbm.at[2 * i + 1, r], ch1, sem1).wait()
            scatter_chunk(ch1)
            pltpu.async_copy(bins_hbm.at[2 * i + 3, r], ch1, sem1)
            return carry
        lax.fori_loop(0, NCH // 2 - 1, cbody, jnp.int32(0))
        pltpu.make_async_copy(bins_hbm.at[NCH - 2, r], ch0, sem0).wait()
        scatter_chunk(ch0)
        pltpu.make_async_copy(bins_hbm.at[NCH - 1, r], ch1, sem1).wait()
        scatter_chunk(ch1)

        def scan_body(i, carry):
            cntv, firstv, runv = carry
            pos0 = i * SCAN_BLK + iota
            hs = [hist[pl.ds(i * SCAN_BLK + d * 16, 16)] for d in range(VPB)]
            css = [plsc.cumsum(h) for h in hs]
            sums = [jnp.sum(h) for h in hs]
            for d in range(VPB):
                hist[pl.ds(i * SCAN_BLK + d * 16, 16)] = zeros16
            offs = [jnp.int32(0)]
            acc = jnp.int32(0)
            for d in range(VPB - 1):
                acc = acc + sums[d]
                offs.append(acc)
            total = acc + sums[VPB - 1]
            for d in range(VPB):
                posv = pos0 + d * 16
                pred = (css[d] + (runv + offs[d]) + c0vec) >= posv + 1
                cntv = cntv + jnp.where(pred, 1, 0)
                firstv = jnp.minimum(firstv, jnp.where(pred, posv, big))
            runv = runv + total
            return (cntv, firstv, runv)
        cntv, firstv, _ = lax.fori_loop(
            0, nblk, scan_body, (zeros16, bigv, zeros16))
        cnt = jnp.sum(cntv)
        first = jnp.min(firstv)

        imp = jnp.where(cnt > 0, (cnt + first).astype(jnp.float32), jnp.float32(0.0))
        res = jnp.where(iota == j, imp, res)

    res_v[...] = res
    pltpu.sync_copy(res_v, out_hbm.at[c * 16 + s])


@functools.partial(jax.jit, static_argnums=())
def _sc_stage(bins, c0, k):
    mesh = plsc.VectorSubcoreMesh(core_axis_name="c", subcore_axis_name="s")
    f = functools.partial(
        pl.kernel,
        out_type=jax.ShapeDtypeStruct((32, 16), jnp.float32),
        mesh=mesh,
        compiler_params=pltpu.CompilerParams(needs_layout_passes=False),
        scratch_types=[
            pltpu.VMEM((N_DIM,), jnp.int32),
            pltpu.VMEM((N_DIM,), jnp.int32),
            pltpu.VMEM((CH,), jnp.int32),
            pltpu.VMEM((CH,), jnp.int32),
            pltpu.VMEM((HIST_WORDS,), jnp.int32),
            pltpu.VMEM((16,), jnp.float32),
            pltpu.SemaphoreType.DMA,
            pltpu.SemaphoreType.DMA,
        ],
    )(_sc_body)
    out = f(bins, c0, k)
    return out[:, :4].reshape(N_DIM)


def kernel(q_mu, q_var):
    bins, c0, k = _tc_stage(q_mu, q_var)
    return _sc_stage(bins, c0.reshape(N_DIM), k.reshape(N_DIM))


# speedup vs baseline: 376.3796x; 1.1537x over previous
import functools

import jax
import jax.numpy as jnp
from jax import lax
from jax.experimental import pallas as pl
from jax.experimental.pallas import tpu as pltpu
from jax.experimental.pallas import tpu_sc as plsc

N_OBJ = 100000
N_DIM = 128
ALPHA = 0.05
SENT = 1 << 20
BN = 2000
GRID = N_OBJ // BN
CH = BN
NCH = N_OBJ // CH
SCAN_BLK = 128
HIST_WORDS = (N_OBJ // SCAN_BLK + 1) * SCAN_BLK


def _tc_body(mu_ref, var_ref, eye_ref, bins_ref, c0_ref, k_ref):
    i = pl.program_id(0)
    mu = mu_ref[...]
    var = var_ref[...]
    z = (0.0 - mu) / (var * jnp.sqrt(jnp.float32(2.0)))
    p = 0.5 * (1.0 + lax.erf(z))
    x = p * jnp.float32(N_OBJ / ALPHA)
    be = x.astype(jnp.int32)
    bef = be.astype(jnp.float32)
    b = be - 1
    for d in range(3):
        tv = ((bef + jnp.float32(d)) / jnp.float32(N_OBJ)) * jnp.float32(ALPHA)
        b = b + jnp.where(tv < p, 1, 0)
    c0_part = jnp.sum(jnp.where(b <= 0, 1, 0), axis=0, keepdims=True)
    k_part = jnp.sum(jnp.where(b < N_OBJ, 1, 0), axis=0, keepdims=True)
    bf = jnp.where(b <= 0, SENT, jnp.minimum(b, N_OBJ)).astype(jnp.float32)
    bT = lax.dot_general(eye_ref[...], bf, (((1,), (1,)), ((), ())),
                         preferred_element_type=jnp.float32)
    bins_ref[...] = bT.astype(jnp.int32).reshape(1, N_DIM, BN)

    @pl.when(i == 0)
    def _():
        c0_ref[...] = c0_part
        k_ref[...] = k_part

    @pl.when(i != 0)
    def _():
        c0_ref[...] += c0_part
        k_ref[...] += k_part


def _tc_stage(q_mu, q_var):
    eye = jnp.eye(N_DIM, dtype=jnp.float32)
    return pl.pallas_call(
        _tc_body,
        grid=(GRID,),
        in_specs=[
            pl.BlockSpec((BN, N_DIM), lambda i: (i, 0)),
            pl.BlockSpec((BN, N_DIM), lambda i: (i, 0)),
            pl.BlockSpec((N_DIM, N_DIM), lambda i: (0, 0)),
        ],
        out_specs=[
            pl.BlockSpec((1, N_DIM, BN), lambda i: (i, 0, 0)),
            pl.BlockSpec((1, N_DIM), lambda i: (0, 0)),
            pl.BlockSpec((1, N_DIM), lambda i: (0, 0)),
        ],
        out_shape=[
            jax.ShapeDtypeStruct((GRID, N_DIM, BN), jnp.int32),
            jax.ShapeDtypeStruct((1, N_DIM), jnp.int32),
            jax.ShapeDtypeStruct((1, N_DIM), jnp.int32),
        ],
    )(q_mu, q_var, eye)


def _sc_body(bins_hbm, c0_hbm, k_hbm, out_hbm,
             kq, c0q, ch0, ch1, ch2, ch3, hist, res_v,
             sem0, sem1, sem2, sem3):
    chs = (ch0, ch1, ch2, ch3)
    sems = (sem0, sem1, sem2, sem3)
    c = lax.axis_index("c")
    s = lax.axis_index("s")
    pltpu.sync_copy(k_hbm, kq)
    pltpu.sync_copy(c0_hbm, c0q)
    iota = lax.iota(jnp.int32, 16)
    zeros16 = jnp.zeros((16,), jnp.int32)
    ones16 = jnp.ones((16,), jnp.int32)
    res = jnp.zeros((16,), jnp.float32)
    big = jnp.int32(1 << 30)
    bigv = jnp.full((16,), big, jnp.int32)
    VPB = SCAN_BLK // 16

    def row_of(j):
        return c * 64 + s * 4 + j

    def dma_start(ci, r, b):
        pltpu.async_copy(bins_hbm.at[ci, r], chs[b], sems[b])

    def dma_wait(ci, r, b):
        pltpu.make_async_copy(bins_hbm.at[ci, r], chs[b], sems[b]).wait()

    for b in range(4):
        dma_start(b, row_of(0), b)

    prev_zeroed = jnp.int32(0)
    for j in range(4):
        r = row_of(j)
        rvec = jnp.full((16,), r, jnp.int32)
        kvec = plsc.load_gather(kq, [rvec])
        c0vec = plsc.load_gather(c0q, [rvec])
        k_sc = jnp.max(kvec)
        nblk = k_sc // SCAN_BLK + 1

        def zbody(i, carry):
            for d in range(VPB):
                hist[pl.ds(i * SCAN_BLK + d * 16, 16)] = zeros16
            return carry
        lax.fori_loop(prev_zeroed, nblk, zbody, 0)
        prev_zeroed = nblk

        def scatter_chunk(buf):
            v1 = buf[pl.ds(0, 16)]
            m1 = v1 < kvec
            v2 = buf[pl.ds(16, 16)]
            m2 = v2 < kvec
            v3 = buf[pl.ds(32, 16)]

            def sbody(i, carry):
                va, ma, vb, mb, vc = carry
                plsc.addupdate_scatter(hist, [va], ones16, mask=ma)
                mc = vc < kvec
                vd = buf[pl.ds(i * 16 + 48, 16)]
                return (vb, mb, vc, mc, vd)
            va, ma, vb, mb, vc = lax.fori_loop(
                0, CH // 16 - 3, sbody, (v1, m1, v2, m2, v3), unroll=5)
            plsc.addupdate_scatter(hist, [va], ones16, mask=ma)
            plsc.addupdate_scatter(hist, [vb], ones16, mask=mb)
            plsc.addupdate_scatter(hist, [vc], ones16, mask=vc < kvec)

        rot = (2 * j) % 4
        order = [(rot + k) % 4 for k in range(4)]

        def gbody(g, carry):
            for k in range(4):
                ci = 4 * g + k
                b = order[k]
                dma_wait(ci, r, b)
                scatter_chunk(chs[b])
                dma_start(ci + 4, r, b)
            return carry
        lax.fori_loop(0, (NCH - 6) // 4, gbody, jnp.int32(0))
        for t in range(6):
            ci = NCH - 6 + t
            b = order[t % 4]
            dma_wait(ci, r, b)
            scatter_chunk(chs[b])
            if t < 2:
                dma_start(ci + 4, r, b)
            elif j < 3:
                dma_start(ci - (NCH - 6 + 2), row_of(j + 1), b)

        def scan_body(i, carry):
            cntv, firstv, runv = carry
            pos0 = i * SCAN_BLK + iota
            hs = [hist[pl.ds(i * SCAN_BLK + d * 16, 16)] for d in range(VPB)]
            css = [plsc.cumsum(h) for h in hs]
            sums = [jnp.sum(h) for h in hs]
            for d in range(VPB):
                hist[pl.ds(i * SCAN_BLK + d * 16, 16)] = zeros16
            offs = [jnp.int32(0)]
            acc = jnp.int32(0)
            for d in range(VPB - 1):
                acc = acc + sums[d]
                offs.append(acc)
            total = acc + sums[VPB - 1]
            for d in range(VPB):
                posv = pos0 + d * 16
                pred = (css[d] + (runv + offs[d]) + c0vec) >= posv + 1
                cntv = cntv + jnp.where(pred, 1, 0)
                firstv = jnp.minimum(firstv, jnp.where(pred, posv, big))
            runv = runv + total
            return (cntv, firstv, runv)
        cntv, firstv, _ = lax.fori_loop(
            0, nblk, scan_body, (zeros16, bigv, zeros16))
        cnt = jnp.sum(cntv)
        first = jnp.min(firstv)

        imp = jnp.where(cnt > 0, (cnt + first).astype(jnp.float32), jnp.float32(0.0))
        res = jnp.where(iota == j, imp, res)

    res_v[...] = res
    pltpu.sync_copy(res_v, out_hbm.at[c * 16 + s])


@functools.partial(jax.jit, static_argnums=())
def _sc_stage(bins, c0, k):
    mesh = plsc.VectorSubcoreMesh(core_axis_name="c", subcore_axis_name="s")
    f = functools.partial(
        pl.kernel,
        out_type=jax.ShapeDtypeStruct((32, 16), jnp.float32),
        mesh=mesh,
        compiler_params=pltpu.CompilerParams(needs_layout_passes=False),
        scratch_types=[
            pltpu.VMEM((N_DIM,), jnp.int32),
            pltpu.VMEM((N_DIM,), jnp.int32),
            pltpu.VMEM((CH,), jnp.int32),
            pltpu.VMEM((CH,), jnp.int32),
            pltpu.VMEM((CH,), jnp.int32),
            pltpu.VMEM((CH,), jnp.int32),
            pltpu.VMEM((HIST_WORDS,), jnp.int32),
            pltpu.VMEM((16,), jnp.float32),
            pltpu.SemaphoreType.DMA,
            pltpu.SemaphoreType.DMA,
            pltpu.SemaphoreType.DMA,
            pltpu.SemaphoreType.DMA,
        ],
    )(_sc_body)
    out = f(bins, c0, k)
    return out[:, :4].reshape(N_DIM)


def kernel(q_mu, q_var):
    bins, c0, k = _tc_stage(q_mu, q_var)
    return _sc_stage(bins, c0.reshape(N_DIM), k.reshape(N_DIM))


# speedup vs baseline: 390.4189x; 1.0373x over previous
import functools

import jax
import jax.numpy as jnp
from jax import lax
from jax.experimental import pallas as pl
from jax.experimental.pallas import tpu as pltpu
from jax.experimental.pallas import tpu_sc as plsc

N_OBJ = 100000
N_DIM = 128
ALPHA = 0.05
SENT = 1 << 20
BN = 2000
GRID = N_OBJ // BN
CH = BN
NCH = N_OBJ // CH
SCAN_BLK = 128
HIST_WORDS = (N_OBJ // SCAN_BLK + 1) * SCAN_BLK


def _tc_body(mu_ref, var_ref, eye_ref, bins_ref, c0_ref, k_ref):
    i = pl.program_id(0)
    mu = mu_ref[...]
    var = var_ref[...]
    z = (0.0 - mu) / (var * jnp.sqrt(jnp.float32(2.0)))
    p = 0.5 * (1.0 + lax.erf(z))
    x = p * jnp.float32(N_OBJ / ALPHA)
    be = x.astype(jnp.int32)
    bef = be.astype(jnp.float32)
    b = be - 1
    for d in range(3):
        tv = ((bef + jnp.float32(d)) / jnp.float32(N_OBJ)) * jnp.float32(ALPHA)
        b = b + jnp.where(tv < p, 1, 0)
    c0_part = jnp.sum(jnp.where(b <= 0, 1, 0), axis=0, keepdims=True)
    k_part = jnp.sum(jnp.where(b < N_OBJ, 1, 0), axis=0, keepdims=True)
    bf = jnp.where(b <= 0, SENT, jnp.minimum(b, N_OBJ)).astype(jnp.float32)
    bT = lax.dot_general(eye_ref[...], bf, (((1,), (1,)), ((), ())),
                         preferred_element_type=jnp.float32)
    bins_ref[...] = bT.astype(jnp.int32).reshape(1, N_DIM, BN)

    @pl.when(i == 0)
    def _():
        c0_ref[...] = c0_part
        k_ref[...] = k_part

    @pl.when(i != 0)
    def _():
        c0_ref[...] += c0_part
        k_ref[...] += k_part


def _tc_stage(q_mu, q_var):
    eye = jnp.eye(N_DIM, dtype=jnp.float32)
    return pl.pallas_call(
        _tc_body,
        grid=(GRID,),
        in_specs=[
            pl.BlockSpec((BN, N_DIM), lambda i: (i, 0)),
            pl.BlockSpec((BN, N_DIM), lambda i: (i, 0)),
            pl.BlockSpec((N_DIM, N_DIM), lambda i: (0, 0)),
        ],
        out_specs=[
            pl.BlockSpec((1, N_DIM, BN), lambda i: (i, 0, 0)),
            pl.BlockSpec((1, N_DIM), lambda i: (0, 0)),
            pl.BlockSpec((1, N_DIM), lambda i: (0, 0)),
        ],
        out_shape=[
            jax.ShapeDtypeStruct((GRID, N_DIM, BN), jnp.int32),
            jax.ShapeDtypeStruct((1, N_DIM), jnp.int32),
            jax.ShapeDtypeStruct((1, N_DIM), jnp.int32),
        ],
    )(q_mu, q_var, eye)


def _sc_body(bins_hbm, c0_hbm, k_hbm, out_hbm,
             kq, c0q, ch0, ch1, ch2, ch3, hist, res_v,
             sem0, sem1, sem2, sem3):
    chs = (ch0, ch1, ch2, ch3)
    sems = (sem0, sem1, sem2, sem3)
    c = lax.axis_index("c")
    s = lax.axis_index("s")
    pltpu.sync_copy(k_hbm, kq)
    pltpu.sync_copy(c0_hbm, c0q)
    iota = lax.iota(jnp.int32, 16)
    zeros16 = jnp.zeros((16,), jnp.int32)
    ones16 = jnp.ones((16,), jnp.int32)
    res = jnp.zeros((16,), jnp.float32)
    big = jnp.int32(1 << 30)
    bigv = jnp.full((16,), big, jnp.int32)
    VPB = SCAN_BLK // 16

    def row_of(j):
        return c * 64 + s * 4 + j

    def dma_start(ci, r, b):
        pltpu.async_copy(bins_hbm.at[ci, r], chs[b], sems[b])

    def dma_wait(ci, r, b):
        pltpu.make_async_copy(bins_hbm.at[ci, r], chs[b], sems[b]).wait()

    for b in range(4):
        dma_start(b, row_of(0), b)

    prev_zeroed = jnp.int32(0)
    for j in range(4):
        r = row_of(j)
        rvec = jnp.full((16,), r, jnp.int32)
        kvec = plsc.load_gather(kq, [rvec])
        c0vec = plsc.load_gather(c0q, [rvec])
        k_sc = jnp.max(kvec)
        nblk = k_sc // SCAN_BLK + 1

        def zbody(i, carry):
            for d in range(VPB):
                hist[pl.ds(i * SCAN_BLK + d * 16, 16)] = zeros16
            return carry
        lax.fori_loop(prev_zeroed, nblk, zbody, 0)
        prev_zeroed = nblk

        def scatter_chunk(buf):
            v1 = buf[pl.ds(0, 16)]
            m1 = v1 < kvec
            v2 = buf[pl.ds(16, 16)]
            m2 = v2 < kvec
            v3 = buf[pl.ds(32, 16)]

            def sbody(i, carry):
                va, ma, vb, mb, vc = carry
                plsc.addupdate_scatter(hist, [va], ones16, mask=ma)
                mc = vc < kvec
                vd = buf[pl.ds(i * 16 + 48, 16)]
                return (vb, mb, vc, mc, vd)
            va, ma, vb, mb, vc = lax.fori_loop(
                0, CH // 16 - 3, sbody, (v1, m1, v2, m2, v3), unroll=10)
            plsc.addupdate_scatter(hist, [va], ones16, mask=ma)
            plsc.addupdate_scatter(hist, [vb], ones16, mask=mb)
            plsc.addupdate_scatter(hist, [vc], ones16, mask=vc < kvec)

        rot = (2 * j) % 4
        order = [(rot + k) % 4 for k in range(4)]

        def gbody(g, carry):
            for k in range(4):
                ci = 4 * g + k
                b = order[k]
                dma_wait(ci, r, b)
                scatter_chunk(chs[b])
                dma_start(ci + 4, r, b)
            return carry
        lax.fori_loop(0, (NCH - 6) // 4, gbody, jnp.int32(0))
        for t in range(6):
            ci = NCH - 6 + t
            b = order[t % 4]
            dma_wait(ci, r, b)
            scatter_chunk(chs[b])
            if t < 2:
                dma_start(ci + 4, r, b)
            elif j < 3:
                dma_start(ci - (NCH - 6 + 2), row_of(j + 1), b)

        def scan_body(i, carry):
            cntv, firstv, runv = carry
            pos0 = i * SCAN_BLK + iota
            hs = [hist[pl.ds(i * SCAN_BLK + d * 16, 16)] for d in range(VPB)]
            css = [plsc.cumsum(h) for h in hs]
            sums = [jnp.sum(h) for h in hs]
            for d in range(VPB):
                hist[pl.ds(i * SCAN_BLK + d * 16, 16)] = zeros16
            offs = [jnp.int32(0)]
            acc = jnp.int32(0)
            for d in range(VPB - 1):
                acc = acc + sums[d]
                offs.append(acc)
            total = acc + sums[VPB - 1]
            for d in range(VPB):
                posv = pos0 + d * 16
                pred = (css[d] + (runv + offs[d]) + c0vec) >= posv + 1
                cntv = cntv + jnp.where(pred, 1, 0)
                firstv = jnp.minimum(firstv, jnp.where(pred, posv, big))
            runv = runv + total
            return (cntv, firstv, runv)
        cntv, firstv, _ = lax.fori_loop(
            0, nblk, scan_body, (zeros16, bigv, zeros16))
        cnt = jnp.sum(cntv)
        first = jnp.min(firstv)

        imp = jnp.where(cnt > 0, (cnt + first).astype(jnp.float32), jnp.float32(0.0))
        res = jnp.where(iota == j, imp, res)

    res_v[...] = res
    pltpu.sync_copy(res_v, out_hbm.at[c * 16 + s])


@functools.partial(jax.jit, static_argnums=())
def _sc_stage(bins, c0, k):
    mesh = plsc.VectorSubcoreMesh(core_axis_name="c", subcore_axis_name="s")
    f = functools.partial(
        pl.kernel,
        out_type=jax.ShapeDtypeStruct((32, 16), jnp.float32),
        mesh=mesh,
        compiler_params=pltpu.CompilerParams(needs_layout_passes=False),
        scratch_types=[
            pltpu.VMEM((N_DIM,), jnp.int32),
            pltpu.VMEM((N_DIM,), jnp.int32),
            pltpu.VMEM((CH,), jnp.int32),
            pltpu.VMEM((CH,), jnp.int32),
            pltpu.VMEM((CH,), jnp.int32),
            pltpu.VMEM((CH,), jnp.int32),
            pltpu.VMEM((HIST_WORDS,), jnp.int32),
            pltpu.VMEM((16,), jnp.float32),
            pltpu.SemaphoreType.DMA,
            pltpu.SemaphoreType.DMA,
            pltpu.SemaphoreType.DMA,
            pltpu.SemaphoreType.DMA,
        ],
    )(_sc_body)
    out = f(bins, c0, k)
    return out[:, :4].reshape(N_DIM)


def kernel(q_mu, q_var):
    bins, c0, k = _tc_stage(q_mu, q_var)
    return _sc_stage(bins, c0.reshape(N_DIM), k.reshape(N_DIM))
